# baseline (speedup 1.0000x reference)
import functools
import math

import jax
import jax.numpy as jnp
from jax import lax
from jax.experimental import pallas as pl
from jax.experimental.pallas import tpu as pltpu
from jax.experimental.pallas import tpu_sc as plsc

TOP_K = 2


def _gate_body(S, E, C, x_ref, wg_ref, idx_ref, w_ref):
    x = x_ref[...]
    wg = wg_ref[...]
    logits = jnp.dot(x, wg, preferred_element_type=jnp.float32)
    m = jnp.max(logits, axis=-1, keepdims=True)
    p = jnp.exp(logits - m)
    gates = p / jnp.sum(p, axis=-1, keepdims=True)

    eids = lax.broadcasted_iota(jnp.int32, (S, E), 1)
    mx1 = jnp.max(gates, axis=-1, keepdims=True)
    idx1 = jnp.min(jnp.where(gates == mx1, eids, E), axis=-1)
    mask1 = eids == idx1[:, None]
    g1 = mx1[:, 0]
    gates2 = jnp.where(mask1, 0.0, gates)
    mx2 = jnp.max(gates2, axis=-1, keepdims=True)
    idx2 = jnp.min(jnp.where(gates2 == mx2, eids, E), axis=-1)
    mask2 = eids == idx2[:, None]
    g2 = mx2[:, 0]

    rows = lax.broadcasted_iota(jnp.int32, (S, S), 0)
    cols = lax.broadcasted_iota(jnp.int32, (S, S), 1)
    tril = (cols <= rows).astype(jnp.float32)
    m1f = mask1.astype(jnp.float32)
    m2f = mask2.astype(jnp.float32)
    cum1 = jnp.dot(tril, m1f, preferred_element_type=jnp.float32)
    cum2 = jnp.dot(tril, m2f, preferred_element_type=jnp.float32)
    cnt1 = jnp.sum(m1f, axis=0, keepdims=True)
    loc1 = (jnp.sum(jnp.where(mask1, cum1, 0.0), axis=-1) - 1.0).astype(jnp.int32)
    loc2 = (jnp.sum(jnp.where(mask2, cum2 + cnt1, 0.0), axis=-1) - 1.0).astype(jnp.int32)

    denom = g1 + g2 + 1e-9
    keep1 = loc1 < C
    keep2 = loc2 < C
    w1 = jnp.where(keep1, g1 / denom, 0.0)
    w2 = jnp.where(keep2, g2 / denom, 0.0)
    srow1 = jnp.where(keep1, idx1 * C + loc1, E * C)
    srow2 = jnp.where(keep2, idx2 * C + loc2, E * C)
    grow1 = idx1 * C + jnp.minimum(loc1, C - 1)
    grow2 = idx2 * C + jnp.minimum(loc2, C - 1)

    idx_ref[0, :] = srow1
    idx_ref[1, :] = srow2
    idx_ref[2, :] = grow1
    idx_ref[3, :] = grow2
    w_ref[0, :, :] = jnp.broadcast_to(w1[:, None], (S, 16))
    w_ref[1, :, :] = jnp.broadcast_to(w2[:, None], (S, 16))


def _ffn_body(d_ref, w1_ref, b1_ref, w2_ref, b2_ref, o_ref):
    d = d_ref[0]
    h = jnp.maximum(
        jnp.dot(d, w1_ref[0], preferred_element_type=jnp.float32) + b1_ref[0],
        0.0)
    o_ref[0] = jnp.dot(h, w2_ref[0], preferred_element_type=jnp.float32) + b2_ref[0]


def _make_dispatch(S, M, n_rows):
    info = plsc.get_sparse_core_info()
    nw = info.num_cores * info.num_subcores
    tok_w = S // nw
    mesh = plsc.VectorSubcoreMesh(core_axis_name="c", subcore_axis_name="s")

    @functools.partial(
        pl.kernel,
        out_type=jax.ShapeDtypeStruct((n_rows, M), jnp.float32),
        mesh=mesh,
        scratch_types=[
            pltpu.VMEM((tok_w, M), jnp.float32),
            pltpu.VMEM((tok_w,), jnp.int32),
            pltpu.VMEM((tok_w,), jnp.int32),
            pltpu.SemaphoreType.DMA,
        ],
    )
    def dispatch(x_hbm, r1_hbm, r2_hbm, out_hbm, src_v, i1_v, i2_v, sem):
        wid = lax.axis_index("s") * info.num_cores + lax.axis_index("c")
        base = wid * tok_w
        pltpu.sync_copy(x_hbm.at[pl.ds(base, tok_w)], src_v)
        pltpu.sync_copy(r1_hbm.at[pl.ds(base, tok_w)], i1_v)
        pltpu.sync_copy(r2_hbm.at[pl.ds(base, tok_w)], i2_v)
        pltpu.async_copy(src_v, out_hbm.at[i1_v], sem).wait()
        pltpu.async_copy(src_v, out_hbm.at[i2_v], sem).wait()

    return dispatch


def _make_combine(S, M, n_rows):
    info = plsc.get_sparse_core_info()
    nw = info.num_cores * info.num_subcores
    tok_w = S // nw
    ht = 32
    rounds = tok_w // ht
    nv = M // 16
    mesh = plsc.VectorSubcoreMesh(core_axis_name="c", subcore_axis_name="s")

    @functools.partial(
        pl.kernel,
        out_type=jax.ShapeDtypeStruct((S, M), jnp.float32),
        mesh=mesh,
        scratch_types=[
            pltpu.VMEM((ht, M), jnp.float32),
            pltpu.VMEM((ht, M), jnp.float32),
            pltpu.VMEM((ht,), jnp.int32),
            pltpu.VMEM((ht,), jnp.int32),
            pltpu.VMEM((ht, 16), jnp.float32),
            pltpu.VMEM((ht, 16), jnp.float32),
            pltpu.SemaphoreType.DMA,
        ],
    )
    def combine(eo_hbm, g1_hbm, g2_hbm, w1_hbm, w2_hbm, out_hbm,
                buf_v, acc_v, i1_v, i2_v, w1_v, w2_v, sem):
        wid = lax.axis_index("s") * info.num_cores + lax.axis_index("c")

        def scaled(t, wref, accumulate):
            wv = wref[t, :]

            def inner(j, _):
                sl = pl.ds(pl.multiple_of(j * 16, 16), 16)
                v = wv * buf_v[t, sl]
                if accumulate:
                    v = acc_v[t, sl] + v
                acc_v[t, sl] = v
                return 0

            lax.fori_loop(0, nv, inner, 0)
            return 0

        for r in range(rounds):
            base = wid * tok_w + r * ht
            pltpu.sync_copy(g1_hbm.at[pl.ds(base, ht)], i1_v)
            pltpu.sync_copy(g2_hbm.at[pl.ds(base, ht)], i2_v)
            pltpu.sync_copy(w1_hbm.at[pl.ds(base, ht)], w1_v)
            pltpu.sync_copy(w2_hbm.at[pl.ds(base, ht)], w2_v)
            pltpu.async_copy(eo_hbm.at[i1_v], buf_v, sem).wait()
            lax.fori_loop(0, ht, lambda t, _: scaled(t, w1_v, False), 0)
            pltpu.async_copy(eo_hbm.at[i2_v], buf_v, sem).wait()
            lax.fori_loop(0, ht, lambda t, _: scaled(t, w2_v, True), 0)
            pltpu.sync_copy(acc_v, out_hbm.at[pl.ds(base, ht)])

    return combine


def kernel(x, Wg, W1, b1, W2, b2):
    B, T, M = x.shape
    S = B * T
    E = Wg.shape[1]
    F = W1.shape[2]
    C = int(math.ceil(TOP_K * S / E))
    xf = x.reshape(S, M)

    idx_out, w_out = pl.pallas_call(
        functools.partial(_gate_body, S, E, C),
        out_shape=[
            jax.ShapeDtypeStruct((4, S), jnp.int32),
            jax.ShapeDtypeStruct((2, S, 16), jnp.float32),
        ],
    )(xf, Wg)

    n_rows = (E + 1) * C
    disp = _make_dispatch(S, M, n_rows)(xf, idx_out[0], idx_out[1])
    disp3 = disp.reshape(E + 1, C, M)

    eo = pl.pallas_call(
        _ffn_body,
        grid=(E,),
        in_specs=[
            pl.BlockSpec((1, C, M), lambda e: (e, 0, 0)),
            pl.BlockSpec((1, M, F), lambda e: (e, 0, 0)),
            pl.BlockSpec((1, 1, F), lambda e: (e, 0, 0)),
            pl.BlockSpec((1, F, M), lambda e: (e, 0, 0)),
            pl.BlockSpec((1, 1, M), lambda e: (e, 0, 0)),
        ],
        out_specs=pl.BlockSpec((1, C, M), lambda e: (e, 0, 0)),
        out_shape=jax.ShapeDtypeStruct((E, C, M), jnp.float32),
        compiler_params=pltpu.CompilerParams(vmem_limit_bytes=100 * 1024 * 1024),
    )(disp3, W1, b1.reshape(E, 1, F), W2, b2.reshape(E, 1, M))

    out = _make_combine(S, M, E * C)(
        eo.reshape(E * C, M), idx_out[2], idx_out[3], w_out[0], w_out[1])
    return out.reshape(B, T, M)


# speedup vs baseline: 1.1836x; 1.1836x over previous
import functools
import math

import jax
import jax.numpy as jnp
from jax import lax
from jax.experimental import pallas as pl
from jax.experimental.pallas import tpu as pltpu
from jax.experimental.pallas import tpu_sc as plsc

TOP_K = 2


def _gate_body(S, E, C, x_ref, wg_ref, idx_ref, w_ref):
    x = x_ref[...]
    wg = wg_ref[...]
    logits = jnp.dot(x, wg, preferred_element_type=jnp.float32)
    m = jnp.max(logits, axis=-1, keepdims=True)
    p = jnp.exp(logits - m)
    gates = p / jnp.sum(p, axis=-1, keepdims=True)

    eids = lax.broadcasted_iota(jnp.int32, (S, E), 1)
    mx1 = jnp.max(gates, axis=-1, keepdims=True)
    idx1 = jnp.min(jnp.where(gates == mx1, eids, E), axis=-1)
    mask1 = eids == idx1[:, None]
    g1 = mx1[:, 0]
    gates2 = jnp.where(mask1, 0.0, gates)
    mx2 = jnp.max(gates2, axis=-1, keepdims=True)
    idx2 = jnp.min(jnp.where(gates2 == mx2, eids, E), axis=-1)
    mask2 = eids == idx2[:, None]
    g2 = mx2[:, 0]

    rows = lax.broadcasted_iota(jnp.int32, (S, S), 0)
    cols = lax.broadcasted_iota(jnp.int32, (S, S), 1)
    tril = (cols <= rows).astype(jnp.float32)
    m1f = mask1.astype(jnp.float32)
    m2f = mask2.astype(jnp.float32)
    cum1 = jnp.dot(tril, m1f, preferred_element_type=jnp.float32)
    cum2 = jnp.dot(tril, m2f, preferred_element_type=jnp.float32)
    cnt1 = jnp.sum(m1f, axis=0, keepdims=True)
    loc1 = (jnp.sum(jnp.where(mask1, cum1, 0.0), axis=-1) - 1.0).astype(jnp.int32)
    loc2 = (jnp.sum(jnp.where(mask2, cum2 + cnt1, 0.0), axis=-1) - 1.0).astype(jnp.int32)

    denom = g1 + g2 + 1e-9
    keep1 = loc1 < C
    keep2 = loc2 < C
    w1 = jnp.where(keep1, g1 / denom, 0.0)
    w2 = jnp.where(keep2, g2 / denom, 0.0)
    srow1 = jnp.where(keep1, idx1 * C + loc1, E * C)
    srow2 = jnp.where(keep2, idx2 * C + loc2, E * C)
    grow1 = idx1 * C + jnp.minimum(loc1, C - 1)
    grow2 = idx2 * C + jnp.minimum(loc2, C - 1)

    idx_ref[0, :] = srow1
    idx_ref[1, :] = srow2
    idx_ref[2, :] = grow1
    idx_ref[3, :] = grow2
    w_ref[0, :, :] = jnp.broadcast_to(w1[:, None], (S, 16))
    w_ref[1, :, :] = jnp.broadcast_to(w2[:, None], (S, 16))


def _ffn_body(d_ref, w1_ref, b1_ref, w2_ref, b2_ref, o_ref):
    d = d_ref[0]
    h = jnp.maximum(
        jnp.dot(d, w1_ref[0], preferred_element_type=jnp.float32) + b1_ref[0],
        0.0)
    o_ref[0] = jnp.dot(h, w2_ref[0], preferred_element_type=jnp.float32) + b2_ref[0]


def _make_dispatch(S, M, n_rows):
    info = plsc.get_sparse_core_info()
    nw = info.num_cores * info.num_subcores
    tok_w = S // nw
    mesh = plsc.VectorSubcoreMesh(core_axis_name="c", subcore_axis_name="s")

    @functools.partial(
        pl.kernel,
        out_type=jax.ShapeDtypeStruct((n_rows, M), jnp.float32),
        mesh=mesh,
        scratch_types=[
            pltpu.VMEM((tok_w, M), jnp.float32),
            pltpu.VMEM((tok_w,), jnp.int32),
            pltpu.VMEM((tok_w,), jnp.int32),
            pltpu.SemaphoreType.DMA,
        ],
    )
    def dispatch(x_hbm, r1_hbm, r2_hbm, out_hbm, src_v, i1_v, i2_v, sem):
        wid = lax.axis_index("s") * info.num_cores + lax.axis_index("c")
        base = wid * tok_w
        pltpu.sync_copy(x_hbm.at[pl.ds(base, tok_w)], src_v)
        pltpu.sync_copy(r1_hbm.at[pl.ds(base, tok_w)], i1_v)
        pltpu.sync_copy(r2_hbm.at[pl.ds(base, tok_w)], i2_v)
        cp1 = pltpu.async_copy(src_v, out_hbm.at[i1_v], sem)
        cp2 = pltpu.async_copy(src_v, out_hbm.at[i2_v], sem)
        cp1.wait()
        cp2.wait()

    return dispatch


def _make_combine(S, M, n_rows):
    info = plsc.get_sparse_core_info()
    nw = info.num_cores * info.num_subcores
    tok_w = S // nw
    ht = 32
    rounds = tok_w // ht
    nv = M // 16
    mesh = plsc.VectorSubcoreMesh(core_axis_name="c", subcore_axis_name="s")

    @functools.partial(
        pl.kernel,
        out_type=jax.ShapeDtypeStruct((S, M), jnp.float32),
        mesh=mesh,
        scratch_types=[
            pltpu.VMEM((ht, M), jnp.float32),
            pltpu.VMEM((ht, M), jnp.float32),
            pltpu.VMEM((ht, M), jnp.float32),
            pltpu.VMEM((ht,), jnp.int32),
            pltpu.VMEM((ht,), jnp.int32),
            pltpu.VMEM((ht, 16), jnp.float32),
            pltpu.VMEM((ht, 16), jnp.float32),
            pltpu.SemaphoreType.DMA,
        ],
    )
    def combine(eo_hbm, g1_hbm, g2_hbm, w1_hbm, w2_hbm, out_hbm,
                buf1_v, buf2_v, acc_v, i1_v, i2_v, w1_v, w2_v, sem):
        wid = lax.axis_index("s") * info.num_cores + lax.axis_index("c")

        for r in range(rounds):
            base = wid * tok_w + r * ht
            pltpu.sync_copy(g1_hbm.at[pl.ds(base, ht)], i1_v)
            pltpu.sync_copy(g2_hbm.at[pl.ds(base, ht)], i2_v)
            pltpu.sync_copy(w1_hbm.at[pl.ds(base, ht)], w1_v)
            pltpu.sync_copy(w2_hbm.at[pl.ds(base, ht)], w2_v)
            cp1 = pltpu.async_copy(eo_hbm.at[i1_v], buf1_v, sem)
            cp2 = pltpu.async_copy(eo_hbm.at[i2_v], buf2_v, sem)
            cp1.wait()
            cp2.wait()

            def body(t, _):
                wv1 = w1_v[t, :]
                wv2 = w2_v[t, :]
                for j in range(nv):
                    sl = pl.ds(j * 16, 16)
                    acc_v[t, sl] = wv1 * buf1_v[t, sl] + wv2 * buf2_v[t, sl]
                return 0

            lax.fori_loop(0, ht, body, 0)
            pltpu.sync_copy(acc_v, out_hbm.at[pl.ds(base, ht)])

    return combine


def kernel(x, Wg, W1, b1, W2, b2):
    B, T, M = x.shape
    S = B * T
    E = Wg.shape[1]
    F = W1.shape[2]
    C = int(math.ceil(TOP_K * S / E))
    xf = x.reshape(S, M)

    idx_out, w_out = pl.pallas_call(
        functools.partial(_gate_body, S, E, C),
        out_shape=[
            jax.ShapeDtypeStruct((4, S), jnp.int32),
            jax.ShapeDtypeStruct((2, S, 16), jnp.float32),
        ],
    )(xf, Wg)

    n_rows = (E + 1) * C
    disp = _make_dispatch(S, M, n_rows)(xf, idx_out[0], idx_out[1])
    disp3 = disp.reshape(E + 1, C, M)

    eo = pl.pallas_call(
        _ffn_body,
        grid=(E,),
        in_specs=[
            pl.BlockSpec((1, C, M), lambda e: (e, 0, 0)),
            pl.BlockSpec((1, M, F), lambda e: (e, 0, 0)),
            pl.BlockSpec((1, 1, F), lambda e: (e, 0, 0)),
            pl.BlockSpec((1, F, M), lambda e: (e, 0, 0)),
            pl.BlockSpec((1, 1, M), lambda e: (e, 0, 0)),
        ],
        out_specs=pl.BlockSpec((1, C, M), lambda e: (e, 0, 0)),
        out_shape=jax.ShapeDtypeStruct((E, C, M), jnp.float32),
        compiler_params=pltpu.CompilerParams(vmem_limit_bytes=100 * 1024 * 1024),
    )(disp3, W1, b1.reshape(E, 1, F), W2, b2.reshape(E, 1, M))

    out = _make_combine(S, M, E * C)(
        eo.reshape(E * C, M), idx_out[2], idx_out[3], w_out[0], w_out[1])
    return out.reshape(B, T, M)
